# baseline (speedup 1.0000x reference)
import functools

import jax
import jax.numpy as jnp
from jax import lax
from jax.experimental import pallas as pl
from jax.experimental.pallas import tpu as pltpu
from jax.experimental.pallas import tpu_sc as plsc

_D = 64
_SEQ = 200
_BATCH = 1024
_SCALE = 8.0

_VOCAB = 1000000
_NC = 2
_NS = 16
_NW = _NC * _NS
_SEQS_PER_W = _BATCH // _NW
_IDX_PER_W = _SEQS_PER_W * _SEQ
_CHUNK_A = 104
_CHUNK_B = _SEQ - _CHUNK_A


def _embed_sc(x_flat, table2, pos):
  mesh = plsc.VectorSubcoreMesh(core_axis_name="c", subcore_axis_name="s")

  @functools.partial(
      pl.kernel,
      out_type=jax.ShapeDtypeStruct((_BATCH * _SEQ, 2 * _D), jnp.float32),
      mesh=mesh,
      compiler_params=pltpu.CompilerParams(use_tc_tiling_on_sc=True,
                                           needs_layout_passes=False),
      scratch_types=[
          pltpu.VMEM((_IDX_PER_W,), jnp.int32),
          pltpu.VMEM((_IDX_PER_W,), jnp.int32),
          pltpu.VMEM((_IDX_PER_W,), jnp.int32),
          pltpu.VMEM((_SEQ, _D), jnp.float32),
          pltpu.VMEM((_SEQ, 2 * _D), jnp.float32),
          pltpu.VMEM((_SEQ, 2 * _D), jnp.float32),
          pltpu.SemaphoreType.DMA,
          pltpu.SemaphoreType.DMA,
          pltpu.SemaphoreType.DMA,
          pltpu.SemaphoreType.DMA,
      ],
  )
  def k(x_hbm, tab_hbm, pos_hbm, out_hbm,
        idx_v, idxj_v, par_v, pos_v, rows0, rows1, sg0, sg1, sw0, sw1):
    w = lax.axis_index("s") * _NC + lax.axis_index("c")
    base = w * _IDX_PER_W

    pltpu.sync_copy(x_hbm.at[pl.ds(base, _IDX_PER_W)], idx_v)
    pltpu.sync_copy(pos_hbm.at[pl.ds(0, _SEQ)], pos_v)

    def prep(i, carry):
      sl = pl.ds(i * 16, 16)
      v = idx_v[sl]
      idxj_v[sl] = lax.shift_right_logical(v, 1)
      par_v[sl] = lax.shift_left(v & 1, 6)
      return carry

    lax.fori_loop(0, _IDX_PER_W // 16, prep, 0)

    rows = (rows0, rows1)
    sg = (sg0, sg1)
    sw = (sw0, sw1)

    def gather_descs(s, p):
      off = s * _SEQ
      return (
          pltpu.make_async_copy(
              tab_hbm.at[idxj_v.at[pl.ds(off, _CHUNK_A)]],
              rows[p].at[pl.ds(0, _CHUNK_A)], sg[p]),
          pltpu.make_async_copy(
              tab_hbm.at[idxj_v.at[pl.ds(off + _CHUNK_A, _CHUNK_B)]],
              rows[p].at[pl.ds(_CHUNK_A, _CHUNK_B)], sg[p]),
      )

    def writeback_desc(s, p):
      out_off = (base + s * _SEQ)
      return pltpu.make_async_copy(
          rows[p], out_hbm.at[pl.ds(out_off, _SEQ)], sw[p])

    col_iota = lax.iota(jnp.int32, 16)

    def compute(s, p):
      rv = rows[p]
      par_base = s * _SEQ

      def body(r, carry):
        rs = lax.broadcast(r, (16,))
        pv = plsc.load_gather(par_v, [rs + par_base])
        for g in range(_D // 16):
          cols = pv + (col_iota + (g * 16))
          val = plsc.load_gather(rv, [rs, cols])
          rv[r, pl.ds(g * 16, 16)] = val * _SCALE + pos_v[r, pl.ds(g * 16, 16)]
        return carry

      lax.fori_loop(0, _SEQ, body, 0)

    for d in gather_descs(0, 0):
      d.start()
    for s in range(_SEQS_PER_W):
      p = s % 2
      if s + 1 < _SEQS_PER_W:
        if s >= 1:
          writeback_desc(s - 1, 1 - p).wait()
        for d in gather_descs(s + 1, 1 - p):
          d.start()
      for d in gather_descs(s, p):
        d.wait()
      compute(s, p)
      writeback_desc(s, p).start()
    writeback_desc(_SEQS_PER_W - 2, 0).wait()
    writeback_desc(_SEQS_PER_W - 1, 1).wait()

  return k


def kernel(x, word_embed_weight, pos_embed_weight):
  x_flat = x.reshape(-1)
  tab2 = word_embed_weight.reshape(_VOCAB // 2, 2 * _D)
  out = _embed_sc(x_flat, tab2, pos_embed_weight)(
      x_flat, tab2, pos_embed_weight)
  return out[:, :_D].reshape(_BATCH, _SEQ, _D)


# speedup vs baseline: 1.0934x; 1.0934x over previous
import functools

import jax
import jax.numpy as jnp
from jax import lax
from jax.experimental import pallas as pl
from jax.experimental.pallas import tpu as pltpu
from jax.experimental.pallas import tpu_sc as plsc

_D = 64
_SEQ = 200
_BATCH = 1024
_SCALE = 8.0

_VOCAB = 1000000
_NC = 2
_NS = 16
_NW = _NC * _NS
_SEQS_PER_W = _BATCH // _NW
_IDX_PER_W = _SEQS_PER_W * _SEQ
_CHUNK_A = 104
_CHUNK_B = _SEQ - _CHUNK_A


def _embed_sc(x_flat, table_pad, pos):
  mesh = plsc.VectorSubcoreMesh(core_axis_name="c", subcore_axis_name="s")

  @functools.partial(
      pl.kernel,
      out_type=jax.ShapeDtypeStruct((_BATCH * _SEQ, 2 * _D), jnp.float32),
      mesh=mesh,
      compiler_params=pltpu.CompilerParams(use_tc_tiling_on_sc=True),
      scratch_types=[
          pltpu.VMEM((_IDX_PER_W,), jnp.int32),
          pltpu.VMEM((_SEQ, _D), jnp.float32),
          pltpu.VMEM((_SEQ, 2 * _D), jnp.float32),
          pltpu.VMEM((_SEQ, 2 * _D), jnp.float32),
          pltpu.SemaphoreType.DMA,
          pltpu.SemaphoreType.DMA,
          pltpu.SemaphoreType.DMA,
          pltpu.SemaphoreType.DMA,
      ],
  )
  def k(x_hbm, tab_hbm, pos_hbm, out_hbm,
        idx_v, pos_v, rows0, rows1, sg0, sg1, sw0, sw1):
    w = lax.axis_index("s") * _NC + lax.axis_index("c")
    base = w * _IDX_PER_W

    pltpu.sync_copy(x_hbm.at[pl.ds(base, _IDX_PER_W)], idx_v)
    pltpu.sync_copy(pos_hbm.at[pl.ds(0, _SEQ)], pos_v)

    rows = (rows0, rows1)
    sg = (sg0, sg1)
    sw = (sw0, sw1)

    def gather_descs(s, p):
      off = s * _SEQ
      return (
          pltpu.make_async_copy(
              tab_hbm.at[idx_v.at[pl.ds(off, _CHUNK_A)]],
              rows[p].at[pl.ds(0, _CHUNK_A)], sg[p]),
          pltpu.make_async_copy(
              tab_hbm.at[idx_v.at[pl.ds(off + _CHUNK_A, _CHUNK_B)]],
              rows[p].at[pl.ds(_CHUNK_A, _CHUNK_B)], sg[p]),
      )

    def writeback_desc(s, p):
      out_off = (base + s * _SEQ)
      return pltpu.make_async_copy(
          rows[p], out_hbm.at[pl.ds(out_off, _SEQ)], sw[p])

    def compute(p):
      rv = rows[p]

      def body(r, carry):
        for g in range(_D // 16):
          sl = pl.ds(g * 16, 16)
          rv[r, sl] = rv[r, sl] * _SCALE + pos_v[r, sl]
        return carry

      lax.fori_loop(0, _SEQ, body, 0)

    for d in gather_descs(0, 0):
      d.start()
    for s in range(_SEQS_PER_W):
      p = s % 2
      if s + 1 < _SEQS_PER_W:
        if s >= 1:
          writeback_desc(s - 1, 1 - p).wait()
        for d in gather_descs(s + 1, 1 - p):
          d.start()
      for d in gather_descs(s, p):
        d.wait()
      compute(p)
      writeback_desc(s, p).start()
    writeback_desc(_SEQS_PER_W - 2, 0).wait()
    writeback_desc(_SEQS_PER_W - 1, 1).wait()

  return k


def kernel(x, word_embed_weight, pos_embed_weight):
  x_flat = x.reshape(-1)
  tab_pad = jnp.concatenate([word_embed_weight, word_embed_weight], axis=1)
  out = _embed_sc(x_flat, tab_pad, pos_embed_weight)(
      x_flat, tab_pad, pos_embed_weight)
  return out[:, :_D].reshape(_BATCH, _SEQ, _D)


# speedup vs baseline: 1.3188x; 1.2062x over previous
import functools

import jax
import jax.numpy as jnp
from jax import lax
from jax.experimental import pallas as pl
from jax.experimental.pallas import tpu as pltpu
from jax.experimental.pallas import tpu_sc as plsc

_D = 64
_SEQ = 200
_BATCH = 1024
_SCALE = 8.0

_VOCAB = 1000000
_NC = 2
_NS = 16
_NW = _NC * _NS
_SEQS_PER_W = _BATCH // _NW
_IDX_PER_W = _SEQS_PER_W * _SEQ
_CHUNK_A = 104
_CHUNK_B = _SEQ - _CHUNK_A


def _embed_sc(x_flat, table_pad, pos):
  mesh = plsc.VectorSubcoreMesh(core_axis_name="c", subcore_axis_name="s")

  @functools.partial(
      pl.kernel,
      out_type=jax.ShapeDtypeStruct((_BATCH * _SEQ, _D), jnp.float32),
      mesh=mesh,
      compiler_params=pltpu.CompilerParams(use_tc_tiling_on_sc=True),
      scratch_types=[
          pltpu.VMEM((_IDX_PER_W,), jnp.int32),
          pltpu.VMEM((_SEQ, _D), jnp.float32),
          pltpu.VMEM((_CHUNK_A, 2 * _D), jnp.float32),
          pltpu.VMEM((_CHUNK_A, 2 * _D), jnp.float32),
          pltpu.VMEM((_CHUNK_A, _D), jnp.float32),
          pltpu.VMEM((_CHUNK_A, _D), jnp.float32),
          pltpu.SemaphoreType.DMA,
          pltpu.SemaphoreType.DMA,
          pltpu.SemaphoreType.DMA,
          pltpu.SemaphoreType.DMA,
      ],
  )
  def k(x_hbm, tab_hbm, pos_hbm, out_hbm,
        idx_v, pos_v, rows0, rows1, st0, st1, sg0, sg1, sw0, sw1):
    w = lax.axis_index("s") * _NC + lax.axis_index("c")
    base = w * _IDX_PER_W

    pltpu.sync_copy(x_hbm.at[pl.ds(base, _IDX_PER_W)], idx_v)
    pltpu.sync_copy(pos_hbm.at[pl.ds(0, _SEQ)], pos_v)

    rows = (rows0, rows1)
    st = (st0, st1)
    sg = (sg0, sg1)
    sw = (sw0, sw1)

    _NSTEP = 2 * _SEQS_PER_W
    def step_off(k):
      return (k // 2) * _SEQ + (k % 2) * _CHUNK_A
    def step_len(k):
      return _CHUNK_A if k % 2 == 0 else _CHUNK_B

    def gather_desc(k, p):
      n = step_len(k)
      return pltpu.make_async_copy(
          tab_hbm.at[idx_v.at[pl.ds(step_off(k), n)]],
          rows[p].at[pl.ds(0, n)], sg[p])

    def writeback_desc(k, p):
      n = step_len(k)
      return pltpu.make_async_copy(
          st[p].at[pl.ds(0, n)],
          out_hbm.at[pl.ds(base + step_off(k), n)], sw[p])

    def compute(k, p):
      rv = rows[p]
      sv = st[p]
      pos_off = (k % 2) * _CHUNK_A

      def body(r, carry):
        for g in range(_D // 16):
          sl = pl.ds(g * 16, 16)
          sv[r, sl] = rv[r, sl] * _SCALE + pos_v[r + pos_off, sl]
        return carry

      lax.fori_loop(0, step_len(k), body, 0)

    gather_desc(0, 0).start()
    for k in range(_NSTEP):
      p = k % 2
      if k + 1 < _NSTEP:
        if k >= 2:
          writeback_desc(k - 1, 1 - p).wait()
        gather_desc(k + 1, 1 - p).start()
      elif k >= 2:
        writeback_desc(k - 1, 1 - p).wait()
      gather_desc(k, p).wait()
      compute(k, p)
      writeback_desc(k, p).start()
    writeback_desc(_NSTEP - 2, 0).wait()
    writeback_desc(_NSTEP - 1, 1).wait()

  return k


def kernel(x, word_embed_weight, pos_embed_weight):
  x_flat = x.reshape(-1)
  tab_pad = jnp.pad(word_embed_weight, ((0, 0), (0, _D)))
  out = _embed_sc(x_flat, tab_pad, pos_embed_weight)(
      x_flat, tab_pad, pos_embed_weight)
  return out.reshape(_BATCH, _SEQ, _D)


# speedup vs baseline: 1.3403x; 1.0163x over previous
import functools

import jax
import jax.numpy as jnp
from jax import lax
from jax.experimental import pallas as pl
from jax.experimental.pallas import tpu as pltpu
from jax.experimental.pallas import tpu_sc as plsc

_D = 64
_SEQ = 200
_BATCH = 1024
_SCALE = 8.0

_VOCAB = 1000000
_NC = 2
_NS = 16
_NW = _NC * _NS
_SEQS_PER_W = _BATCH // _NW
_IDX_PER_W = _SEQS_PER_W * _SEQ
_CHUNK_A = 104
_CHUNK_B = _SEQ - _CHUNK_A


def _embed_sc(x_flat, table_pad, pos):
  mesh = plsc.VectorSubcoreMesh(core_axis_name="c", subcore_axis_name="s")

  @functools.partial(
      pl.kernel,
      out_type=jax.ShapeDtypeStruct((_BATCH * _SEQ, 2 * _D), jnp.float32),
      mesh=mesh,
      compiler_params=pltpu.CompilerParams(use_tc_tiling_on_sc=True),
      scratch_types=[
          pltpu.VMEM((_IDX_PER_W,), jnp.int32),
          pltpu.VMEM((_SEQ, _D), jnp.float32),
          pltpu.VMEM((_SEQ, 2 * _D), jnp.float32),
          pltpu.VMEM((_SEQ, 2 * _D), jnp.float32),
          pltpu.SemaphoreType.DMA,
          pltpu.SemaphoreType.DMA,
          pltpu.SemaphoreType.DMA,
          pltpu.SemaphoreType.DMA,
      ],
  )
  def k(x_hbm, tab_hbm, pos_hbm, out_hbm,
        idx_v, pos_v, rows0, rows1, sg0, sg1, sw0, sw1):
    w = lax.axis_index("s") * _NC + lax.axis_index("c")
    base = w * _IDX_PER_W

    pltpu.sync_copy(x_hbm.at[pl.ds(base, _IDX_PER_W)], idx_v)
    pltpu.sync_copy(pos_hbm.at[pl.ds(0, _SEQ)], pos_v)

    rows = (rows0, rows1)
    sg = (sg0, sg1)
    sw = (sw0, sw1)

    def gather_descs(s, p):
      off = s * _SEQ
      return (
          pltpu.make_async_copy(
              tab_hbm.at[idx_v.at[pl.ds(off, _CHUNK_A)]],
              rows[p].at[pl.ds(0, _CHUNK_A)], sg[p]),
          pltpu.make_async_copy(
              tab_hbm.at[idx_v.at[pl.ds(off + _CHUNK_A, _CHUNK_B)]],
              rows[p].at[pl.ds(_CHUNK_A, _CHUNK_B)], sg[p]),
      )

    def writeback_desc(s, p):
      out_off = (base + s * _SEQ)
      return pltpu.make_async_copy(
          rows[p], out_hbm.at[pl.ds(out_off, _SEQ)], sw[p])

    def compute(p):
      rv = rows[p]

      def body(r, carry):
        for g in range(_D // 16):
          sl = pl.ds(g * 16, 16)
          rv[r, sl] = rv[r, sl] * _SCALE + pos_v[r, sl]
        return carry

      lax.fori_loop(0, _SEQ, body, 0)

    for d in gather_descs(0, 0):
      d.start()
    for s in range(_SEQS_PER_W):
      p = s % 2
      if s + 1 < _SEQS_PER_W:
        if s >= 1:
          writeback_desc(s - 1, 1 - p).wait()
        for d in gather_descs(s + 1, 1 - p):
          d.start()
      for d in gather_descs(s, p):
        d.wait()
      compute(p)
      writeback_desc(s, p).start()
    writeback_desc(_SEQS_PER_W - 2, 0).wait()
    writeback_desc(_SEQS_PER_W - 1, 1).wait()

  return k


def kernel(x, word_embed_weight, pos_embed_weight):
  x_flat = x.reshape(-1)
  tab_pad = jnp.pad(word_embed_weight, ((0, 0), (0, _D)))
  out = _embed_sc(x_flat, tab_pad, pos_embed_weight)(
      x_flat, tab_pad, pos_embed_weight)
  return out[:, :_D].reshape(_BATCH, _SEQ, _D)
